# baseline (speedup 1.0000x reference)
import functools

import jax
import jax.numpy as jnp
from jax import lax
from jax.experimental import pallas as pl
from jax.experimental.pallas import tpu as pltpu
from jax.experimental.pallas import tpu_sc as plsc

_E = 64
_EP = 128
_BLK = 128
_SKEW = 144
_NW = 32


@functools.partial(jax.jit, static_argnames=("n_t", "n_b"))
def _gather_t_major(idx_t, table_p, n_t, n_b):
    mesh = plsc.VectorSubcoreMesh(core_axis_name="c", subcore_axis_name="s")

    @functools.partial(
        pl.kernel,
        mesh=mesh,
        out_type=jax.ShapeDtypeStruct((n_t, _E, n_b), jnp.float32),
        scratch_types=[
            pltpu.VMEM((n_t, _BLK), jnp.int32),
            pltpu.VMEM((_BLK, _EP), jnp.float32),
            pltpu.VMEM((_BLK, _EP), jnp.float32),
            pltpu.VMEM((1, _E, _SKEW), jnp.float32),
            pltpu.VMEM((1, _E, _SKEW), jnp.float32),
            pltpu.SemaphoreType.DMA,
            pltpu.SemaphoreType.DMA,
            pltpu.SemaphoreType.DMA,
            pltpu.SemaphoreType.DMA,
        ],
        compiler_params=pltpu.CompilerParams(
            use_tc_tiling_on_sc=True, needs_layout_passes=False
        ),
    )
    def emb_kernel(idx_hbm, table_hbm, out_hbm,
                   idx_v, rows0, rows1, outb0, outb1, g0, g1, w0, w1):
        wid = lax.axis_index("s") * 2 + lax.axis_index("c")
        b0 = wid * _BLK
        rows = [rows0, rows1]
        outb = [outb0, outb1]
        gsem = [g0, g1]
        wsem = [w0, w1]

        pltpu.sync_copy(idx_hbm.at[:, pl.ds(b0, _BLK)], idx_v)

        def gather_desc(t, rb):
            return pltpu.make_async_copy(
                table_hbm.at[idx_v.at[t]], rows[rb], gsem[rb]
            )

        def write_desc(t, rb):
            return pltpu.make_async_copy(
                outb[rb].at[:, :, pl.ds(0, _BLK)],
                out_hbm.at[pl.ds(t, 1), :, pl.ds(b0, _BLK)],
                wsem[rb],
            )

        def repack(rb):
            src = rows[rb]
            dst = outb[rb]
            zero = jnp.zeros((16,), jnp.int32)
            e_vecs = [
                lax.iota(jnp.int32, 16) + (16 * eg) for eg in range(_E // 16)
            ]
            @plsc.parallel_loop(0, _BLK, unroll=2)
            def bbody(b):
                b_vec = jnp.full((16,), 0, jnp.int32) + b
                vals = [
                    src[b, pl.ds(16 * eg, 16)] for eg in range(_E // 16)
                ]
                for eg in range(_E // 16):
                    plsc.store_scatter(
                        dst, [zero, e_vecs[eg], b_vec], vals[eg]
                    )

        gather_desc(0, 0).start()

        def step(t, rb):
            @pl.when(t + 1 < n_t)
            def _():
                gather_desc(t + 1, 1 - rb).start()

            gather_desc(t, rb).wait()

            @pl.when(t >= 2)
            def _():
                write_desc(t - 2, rb).wait()

            repack(rb)
            write_desc(t, rb).start()

        def pair(i, carry):
            step(2 * i, 0)
            step(2 * i + 1, 1)
            return carry

        lax.fori_loop(0, n_t // 2, pair, 0)
        write_desc(n_t - 2, 0).wait()
        write_desc(n_t - 1, 1).wait()

    return emb_kernel(idx_t, table_p)


def kernel(tensor, embedding_weight):
    batch, hist = tensor.shape
    idx_t = jnp.swapaxes(tensor.astype(jnp.int32), 0, 1)
    table_p = jnp.pad(embedding_weight, ((0, 0), (0, _EP - _E)))
    out3 = _gather_t_major(idx_t, table_p, hist, batch)
    return jnp.transpose(out3, (2, 0, 1))


# speedup vs baseline: 1.1273x; 1.1273x over previous
import functools

import jax
import jax.numpy as jnp
from jax import lax
from jax.experimental import pallas as pl
from jax.experimental.pallas import tpu as pltpu
from jax.experimental.pallas import tpu_sc as plsc

_EMBED = 64
_NUM_WORKERS = 32
_CHUNK = 512
_NBUF = 2


@functools.partial(jax.jit, static_argnames=("total",))
def _gather_flat(idx_flat, table, total):
    b_per_w = total // _NUM_WORKERS
    n_chunks = b_per_w // _CHUNK
    mesh = plsc.VectorSubcoreMesh(core_axis_name="c", subcore_axis_name="s")

    @functools.partial(
        pl.kernel,
        mesh=mesh,
        out_type=jax.ShapeDtypeStruct((total, _EMBED), jnp.float32),
        scratch_types=[
            pltpu.VMEM((b_per_w,), jnp.int32),
            pltpu.VMEM((_NBUF, _CHUNK, _EMBED), jnp.float32),
            pltpu.SemaphoreType.DMA,
            pltpu.SemaphoreType.DMA,
            pltpu.SemaphoreType.DMA,
            pltpu.SemaphoreType.DMA,
        ],
        compiler_params=pltpu.CompilerParams(use_tc_tiling_on_sc=False),
    )
    def emb_kernel(idx_hbm, table_hbm, out_hbm, idx_v, rows_v, g0, g1, w0, w1):
        wid = lax.axis_index("s") * 2 + lax.axis_index("c")
        base = wid * b_per_w
        gsem = [g0, g1]
        wsem = [w0, w1]
        pltpu.sync_copy(idx_hbm.at[pl.ds(base, b_per_w)], idx_v)

        def gather_desc(g, b):
            return pltpu.make_async_copy(
                table_hbm.at[idx_v.at[pl.ds(g * _CHUNK, _CHUNK)]],
                rows_v.at[b],
                gsem[b],
            )

        def write_desc(g, b):
            return pltpu.make_async_copy(
                rows_v.at[b],
                out_hbm.at[pl.ds(base + g * _CHUNK, _CHUNK)],
                wsem[b],
            )

        gather_desc(0, 0).start()

        def step(g, b):
            nb = 1 - b

            @pl.when(g + 1 < n_chunks)
            def _():
                @pl.when(g >= 1)
                def _():
                    write_desc(g - 1, nb).wait()

                gather_desc(g + 1, nb).start()

            gather_desc(g, b).wait()
            write_desc(g, b).start()

        def pair(t, carry):
            step(2 * t, 0)
            step(2 * t + 1, 1)
            return carry

        lax.fori_loop(0, n_chunks // 2, pair, 0)
        write_desc(n_chunks - 2, (n_chunks - 2) % 2).wait()
        write_desc(n_chunks - 1, (n_chunks - 1) % 2).wait()

    return emb_kernel(idx_flat, table)


def kernel(tensor, embedding_weight):
    batch, hist = tensor.shape
    total = batch * hist
    idx_flat = tensor.reshape(total).astype(jnp.int32)
    out = _gather_flat(idx_flat, embedding_weight, total)
    return out.reshape(batch, hist, _EMBED)
